# baseline (speedup 1.0000x reference)
import jax
import jax.numpy as jnp
from jax.experimental import pallas as pl


def _dot_nt_3pass(a, b):
    ah = a.astype(jnp.bfloat16)
    al = (a - ah.astype(jnp.float32)).astype(jnp.bfloat16)
    bh = b.astype(jnp.bfloat16)
    bl = (b - bh.astype(jnp.float32)).astype(jnp.bfloat16)
    dims = (((1,), (1,)), ((), ()))
    hh = jax.lax.dot_general(ah, bh, dims, preferred_element_type=jnp.float32)
    hl = jax.lax.dot_general(ah, bl, dims, preferred_element_type=jnp.float32)
    lh = jax.lax.dot_general(al, bh, dims, preferred_element_type=jnp.float32)
    return hh + (hl + lh)


def _codebook_kernel(x_ref, c_ref, out_ref):
    _, _, h_sz, w_sz = out_ref.shape
    xb = x_ref[...]
    cb = c_ref[...]
    scores = _dot_nt_3pass(xb, cb)
    half_norm = 0.5 * jnp.sum(cb * cb, axis=1)[None, :]
    scores = scores - half_norm
    k = scores.shape[1]
    best = jnp.max(scores, axis=1, keepdims=True)
    iota = jax.lax.broadcasted_iota(jnp.int32, scores.shape, 1)
    idx = jnp.min(jnp.where(scores == best, iota, k), axis=1)
    out_ref[0, 0] = jnp.stack(
        [idx[h * w_sz:(h + 1) * w_sz] for h in range(h_sz)])


def kernel(x, cluster_centers):
    b, c, h, w = x.shape
    k = cluster_centers.shape[1]
    p = b * h * w
    xp = jnp.transpose(x, (0, 2, 3, 1)).reshape(p, c)
    cc = cluster_centers.reshape(k, c)

    return pl.pallas_call(
        _codebook_kernel,
        grid=(b,),
        in_specs=[
            pl.BlockSpec((h * w, c), lambda i: (i, 0)),
            pl.BlockSpec((k, c), lambda i: (0, 0)),
        ],
        out_specs=pl.BlockSpec((1, 1, h, w), lambda i: (i, 0, 0, 0)),
        out_shape=jax.ShapeDtypeStruct((b, 1, h, w), jnp.int32),
    )(xp, cc)


# speedup vs baseline: 1.0915x; 1.0915x over previous
import jax
import jax.numpy as jnp
from jax.experimental import pallas as pl


def _dot_nt_3pass(a, b):
    ah = a.astype(jnp.bfloat16)
    al = (a - ah.astype(jnp.float32)).astype(jnp.bfloat16)
    bh = b.astype(jnp.bfloat16)
    bl = (b - bh.astype(jnp.float32)).astype(jnp.bfloat16)
    dims = (((1,), (1,)), ((), ()))
    hh = jax.lax.dot_general(ah, bh, dims, preferred_element_type=jnp.float32)
    hl = jax.lax.dot_general(ah, bl, dims, preferred_element_type=jnp.float32)
    lh = jax.lax.dot_general(al, bh, dims, preferred_element_type=jnp.float32)
    return hh + (hl + lh)


def _codebook_kernel(x_ref, c_ref, out_ref):
    b_sz, _, h_sz, w_sz = out_ref.shape
    xb = x_ref[...]
    cb = c_ref[...]
    scores = _dot_nt_3pass(xb, cb)
    half_norm = 0.5 * jnp.sum(cb * cb, axis=1)[None, :]
    scores = scores - half_norm
    k = scores.shape[1]
    best = jnp.max(scores, axis=1, keepdims=True)
    iota = jax.lax.broadcasted_iota(jnp.int32, (1, k), 1).astype(jnp.float32)
    idx = jnp.min(jnp.where(scores == best, iota, float(k)),
                  axis=1).astype(jnp.int32)
    for b in range(b_sz):
        out_ref[b, 0] = jnp.stack(
            [idx[(b * h_sz + h) * w_sz:(b * h_sz + h + 1) * w_sz]
             for h in range(h_sz)])


def kernel(x, cluster_centers):
    b, c, h, w = x.shape
    k = cluster_centers.shape[1]
    p = b * h * w
    xp = jnp.transpose(x, (0, 2, 3, 1)).reshape(p, c)
    cc = cluster_centers.reshape(k, c)

    return pl.pallas_call(
        _codebook_kernel,
        out_shape=jax.ShapeDtypeStruct((b, 1, h, w), jnp.int32),
    )(xp, cc)
